# baseline (speedup 1.0000x reference)
import functools

import jax
import jax.numpy as jnp
from jax import lax
from jax.experimental import pallas as pl
from jax.experimental.pallas import tpu as pltpu
from jax.experimental.pallas import tpu_sc as plsc

NUM_ENTITIES = 1000000
NUM_RELATIONS = 1000
D = 32
PD = 128
B = 16384
NC = 2
NS = 16
L = 16
NW = NC * NS
BPW = B // NW
CH = 128
NCH = BPW // CH
GPC = CH // L

EBLK = 2048
NBLK = -(-NUM_ENTITIES // EBLK)
RBLK = EBLK // 4


def _repack_body(x_ref, o_ref):
    eye = jnp.eye(D, dtype=jnp.float32)
    outs = []
    for k in range(4):
        xk = x_ref[:, pl.ds(512 * k, 512)]
        outs.append(jax.lax.dot_general(
            xk, eye, (((0,), (0,)), ((), ())),
            preferred_element_type=jnp.float32))
    o_ref[...] = jnp.concatenate(outs, axis=1)


_repack = pl.pallas_call(
    _repack_body,
    grid=(NBLK,),
    in_specs=[pl.BlockSpec((D, EBLK), lambda i: (0, i))],
    out_specs=pl.BlockSpec((RBLK, PD), lambda i: (i, 0)),
    out_shape=jax.ShapeDtypeStruct((NBLK * RBLK, PD), jnp.float32),
)

_mesh = plsc.VectorSubcoreMesh(core_axis_name="c", subcore_axis_name="s")


@functools.partial(
    pl.kernel,
    out_type=jax.ShapeDtypeStruct((B,), jnp.float32),
    mesh=_mesh,
    compiler_params=pltpu.CompilerParams(
        needs_layout_passes=False,
    ),
    scratch_types=[
        pltpu.VMEM((BPW,), jnp.int32),
        pltpu.VMEM((BPW,), jnp.int32),
        pltpu.VMEM((BPW,), jnp.int32),
        pltpu.VMEM((NCH, CH), jnp.int32),
        pltpu.VMEM((NCH, CH), jnp.int32),
        pltpu.VMEM((NCH, CH), jnp.int32),
        pltpu.VMEM((2, CH, PD), jnp.float32),
        pltpu.VMEM((2, CH, PD), jnp.float32),
        pltpu.VMEM((2, CH, PD), jnp.float32),
        pltpu.VMEM((BPW,), jnp.float32),
        pltpu.SemaphoreType.DMA,
        pltpu.SemaphoreType.DMA,
    ],
)
def _transe_scores(ent4_h, relp_h, hi_h, ri_h, ti_h, out_h,
                   hiv, riv, tiv, hrow, rrow, trow,
                   hbuf, rbuf, tbuf, outv, sem0, sem1):
    wid = lax.axis_index("s") * NC + lax.axis_index("c")
    base = wid * BPW

    pltpu.sync_copy(hi_h.at[pl.ds(base, BPW)], hiv)
    pltpu.sync_copy(ri_h.at[pl.ds(base, BPW)], riv)
    pltpu.sync_copy(ti_h.at[pl.ds(base, BPW)], tiv)

    for q in range(NCH):
        for k in range(CH // L):
            sl = pl.ds(q * CH + k * L, L)
            dsl = pl.ds(k * L, L)
            he = hiv[sl]
            te = tiv[sl]
            hrow[q, dsl] = ((he >> 11) << 9) + (he & 511)
            rrow[q, dsl] = riv[sl]
            trow[q, dsl] = ((te >> 11) << 9) + (te & 511)

    sems = (sem0, sem1)

    def issue(q):
        s = sems[q % 2]
        b = q % 2
        return [
            pltpu.async_copy(ent4_h.at[hrow.at[q]], hbuf.at[b], s),
            pltpu.async_copy(ent4_h.at[trow.at[q]], tbuf.at[b], s),
            pltpu.async_copy(relp_h.at[rrow.at[q]], rbuf.at[b], s),
        ]

    lanes = lax.iota(jnp.int32, L)
    pending = issue(0)
    for q in range(NCH):
        for c in pending:
            c.wait()
        if q + 1 < NCH:
            pending = issue(q + 1)
        b = q % 2

        def group(g, carry, _q=q, _b=b):
            rows = g * L + lanes
            gsl = pl.ds(_q * CH + g * L, L)
            hoff = ((hiv[gsl] >> 9) & 3) * D
            toff = ((tiv[gsl] >> 9) & 3) * D
            acc = jnp.zeros((L,), jnp.float32)
            for j in range(D):
                cols = jnp.full((L,), j, jnp.int32)
                hv = plsc.load_gather(hbuf.at[_b], [rows, hoff + j])
                rv = plsc.load_gather(rbuf.at[_b], [rows, cols])
                tv = plsc.load_gather(tbuf.at[_b], [rows, toff + j])
                d = (hv + rv) - tv
                acc = acc + d * d
            i32 = plsc.bitcast(acc, jnp.int32)
            y = plsc.bitcast(
                jnp.full((L,), 0x5F3759DF, jnp.int32) - (i32 >> 1),
                jnp.float32)
            for _ in range(3):
                y = y * (1.5 - 0.5 * ((acc * y) * y))
            outv[pl.ds(_q * CH + g * L, L)] = -(acc * y)
            return carry

        lax.fori_loop(0, GPC, group, 0)

    pltpu.sync_copy(outv, out_h.at[pl.ds(base, BPW)])


def kernel(entity_embeddings, relation_embeddings,
           head_indices, relation_indices, tail_indices):
    ent4 = _repack(entity_embeddings.T)
    relp = jnp.pad(relation_embeddings, ((0, 0), (0, PD - D)))
    return _transe_scores(ent4, relp,
                          head_indices, relation_indices, tail_indices)


# speedup vs baseline: 1.5530x; 1.5530x over previous
import functools

import jax
import jax.numpy as jnp
from jax import lax
from jax.experimental import pallas as pl
from jax.experimental.pallas import tpu as pltpu
from jax.experimental.pallas import tpu_sc as plsc

NUM_ENTITIES = 1000000
NUM_RELATIONS = 1000
D = 32
PD = 128
B = 16384
NC = 2
NS = 16
L = 16
NW = NC * NS
BPW = B // NW
CH = 128
NCH = BPW // CH
GPC = CH // L

EBLK = 8192
NBLK = -(-NUM_ENTITIES // EBLK)
RBLK = EBLK // 4


def _repack_body(x_ref, o_ref):
    eye = jnp.eye(D, dtype=jnp.float32)
    outs = []
    for k in range(4):
        xk = x_ref[:, pl.ds(2048 * k, 2048)]
        outs.append(jax.lax.dot_general(
            xk, eye, (((0,), (0,)), ((), ())),
            preferred_element_type=jnp.float32))
    o_ref[...] = jnp.concatenate(outs, axis=1)


_repack = pl.pallas_call(
    _repack_body,
    grid=(NBLK,),
    in_specs=[pl.BlockSpec((D, EBLK), lambda i: (0, i))],
    out_specs=pl.BlockSpec((RBLK, PD), lambda i: (i, 0)),
    out_shape=jax.ShapeDtypeStruct((NBLK * RBLK, PD), jnp.float32),
)

_mesh = plsc.VectorSubcoreMesh(core_axis_name="c", subcore_axis_name="s")


@functools.partial(
    pl.kernel,
    out_type=jax.ShapeDtypeStruct((B,), jnp.float32),
    mesh=_mesh,
    compiler_params=pltpu.CompilerParams(
        needs_layout_passes=False,
    ),
    scratch_types=[
        pltpu.VMEM((BPW,), jnp.int32),
        pltpu.VMEM((BPW,), jnp.int32),
        pltpu.VMEM((BPW,), jnp.int32),
        pltpu.VMEM((NCH, CH), jnp.int32),
        pltpu.VMEM((NCH, CH), jnp.int32),
        pltpu.VMEM((NCH, CH), jnp.int32),
        pltpu.VMEM((2, CH, PD), jnp.float32),
        pltpu.VMEM((2, CH, PD), jnp.float32),
        pltpu.VMEM((2, CH, PD), jnp.float32),
        pltpu.VMEM((BPW,), jnp.float32),
        pltpu.SemaphoreType.DMA,
        pltpu.SemaphoreType.DMA,
    ],
)
def _transe_scores(ent4_h, relp_h, hi_h, ri_h, ti_h, out_h,
                   hiv, riv, tiv, hrow, rrow, trow,
                   hbuf, rbuf, tbuf, outv, sem0, sem1):
    wid = lax.axis_index("s") * NC + lax.axis_index("c")
    base = wid * BPW

    pltpu.sync_copy(hi_h.at[pl.ds(base, BPW)], hiv)
    pltpu.sync_copy(ri_h.at[pl.ds(base, BPW)], riv)
    pltpu.sync_copy(ti_h.at[pl.ds(base, BPW)], tiv)

    for q in range(NCH):
        for k in range(CH // L):
            sl = pl.ds(q * CH + k * L, L)
            dsl = pl.ds(k * L, L)
            he = hiv[sl]
            te = tiv[sl]
            hrow[q, dsl] = ((he >> 13) << 11) + (he & 2047)
            rrow[q, dsl] = riv[sl]
            trow[q, dsl] = ((te >> 13) << 11) + (te & 2047)

    sems = (sem0, sem1)

    def issue(q):
        s = sems[q % 2]
        b = q % 2
        return [
            pltpu.async_copy(ent4_h.at[hrow.at[q]], hbuf.at[b], s),
            pltpu.async_copy(ent4_h.at[trow.at[q]], tbuf.at[b], s),
            pltpu.async_copy(relp_h.at[rrow.at[q]], rbuf.at[b], s),
        ]

    lanes = lax.iota(jnp.int32, L)
    pending = issue(0)
    for q in range(NCH):
        for c in pending:
            c.wait()
        if q + 1 < NCH:
            pending = issue(q + 1)
        b = q % 2

        def group(g, carry, _q=q, _b=b):
            rows = g * L + lanes
            gsl = pl.ds(_q * CH + g * L, L)
            hoff = ((hiv[gsl] >> 11) & 3) * D
            toff = ((tiv[gsl] >> 11) & 3) * D
            acc = jnp.zeros((L,), jnp.float32)
            for j in range(D):
                cols = jnp.full((L,), j, jnp.int32)
                hv = plsc.load_gather(hbuf.at[_b], [rows, hoff + j])
                rv = plsc.load_gather(rbuf.at[_b], [rows, cols])
                tv = plsc.load_gather(tbuf.at[_b], [rows, toff + j])
                d = (hv + rv) - tv
                acc = acc + d * d
            i32 = plsc.bitcast(acc, jnp.int32)
            y = plsc.bitcast(
                jnp.full((L,), 0x5F3759DF, jnp.int32) - (i32 >> 1),
                jnp.float32)
            for _ in range(3):
                y = y * (1.5 - 0.5 * ((acc * y) * y))
            outv[pl.ds(_q * CH + g * L, L)] = -(acc * y)
            return carry

        lax.fori_loop(0, GPC, group, 0)

    pltpu.sync_copy(outv, out_h.at[pl.ds(base, BPW)])


def kernel(entity_embeddings, relation_embeddings,
           head_indices, relation_indices, tail_indices):
    ent4 = _repack(entity_embeddings.T)
    relp = jnp.pad(relation_embeddings, ((0, 0), (0, PD - D)))
    return _transe_scores(ent4, relp,
                          head_indices, relation_indices, tail_indices)


# speedup vs baseline: 1.5852x; 1.0207x over previous
import functools

import jax
import jax.numpy as jnp
from jax import lax
from jax.experimental import pallas as pl
from jax.experimental.pallas import tpu as pltpu
from jax.experimental.pallas import tpu_sc as plsc

NUM_ENTITIES = 1000000
NUM_RELATIONS = 1000
D = 32
PD = 128
B = 16384
NC = 2
NS = 16
L = 16
NW = NC * NS
BPW = B // NW
CH = 128
NCH = BPW // CH
GPC = CH // L

EBLK = 32768
NBLK = -(-NUM_ENTITIES // EBLK)
RBLK = EBLK // 4


def _repack_body(x_ref, o_ref):
    eye = jnp.eye(D, dtype=jnp.float32)
    outs = []
    for k in range(4):
        xk = x_ref[:, pl.ds((EBLK // 4) * k, EBLK // 4)]
        outs.append(jax.lax.dot_general(
            xk, eye, (((0,), (0,)), ((), ())),
            preferred_element_type=jnp.float32))
    o_ref[...] = jnp.concatenate(outs, axis=1)


_repack = pl.pallas_call(
    _repack_body,
    grid=(NBLK,),
    in_specs=[pl.BlockSpec((D, EBLK), lambda i: (0, i))],
    out_specs=pl.BlockSpec((RBLK, PD), lambda i: (i, 0)),
    out_shape=jax.ShapeDtypeStruct((NBLK * RBLK, PD), jnp.float32),
)

_mesh = plsc.VectorSubcoreMesh(core_axis_name="c", subcore_axis_name="s")


@functools.partial(
    pl.kernel,
    out_type=jax.ShapeDtypeStruct((B,), jnp.float32),
    mesh=_mesh,
    compiler_params=pltpu.CompilerParams(
        needs_layout_passes=False,
    ),
    scratch_types=[
        pltpu.VMEM((BPW,), jnp.int32),
        pltpu.VMEM((BPW,), jnp.int32),
        pltpu.VMEM((BPW,), jnp.int32),
        pltpu.VMEM((NCH, CH), jnp.int32),
        pltpu.VMEM((NCH, CH), jnp.int32),
        pltpu.VMEM((NCH, CH), jnp.int32),
        pltpu.VMEM((2, CH, PD), jnp.float32),
        pltpu.VMEM((2, CH, PD), jnp.float32),
        pltpu.VMEM((2, CH, PD), jnp.float32),
        pltpu.VMEM((BPW,), jnp.float32),
        pltpu.SemaphoreType.DMA,
        pltpu.SemaphoreType.DMA,
    ],
)
def _transe_scores(ent4_h, relp_h, hi_h, ri_h, ti_h, out_h,
                   hiv, riv, tiv, hrow, rrow, trow,
                   hbuf, rbuf, tbuf, outv, sem0, sem1):
    wid = lax.axis_index("s") * NC + lax.axis_index("c")
    base = wid * BPW

    pltpu.sync_copy(hi_h.at[pl.ds(base, BPW)], hiv)
    pltpu.sync_copy(ri_h.at[pl.ds(base, BPW)], riv)
    pltpu.sync_copy(ti_h.at[pl.ds(base, BPW)], tiv)

    for q in range(NCH):
        for k in range(CH // L):
            sl = pl.ds(q * CH + k * L, L)
            dsl = pl.ds(k * L, L)
            he = hiv[sl]
            te = tiv[sl]
            hrow[q, dsl] = ((he >> 15) << 13) + (he & 8191)
            rrow[q, dsl] = riv[sl]
            trow[q, dsl] = ((te >> 15) << 13) + (te & 8191)

    sems = (sem0, sem1)

    def issue(q):
        s = sems[q % 2]
        b = q % 2
        return [
            pltpu.async_copy(ent4_h.at[hrow.at[q]], hbuf.at[b], s),
            pltpu.async_copy(ent4_h.at[trow.at[q]], tbuf.at[b], s),
            pltpu.async_copy(relp_h.at[rrow.at[q]], rbuf.at[b], s),
        ]

    lanes = lax.iota(jnp.int32, L)
    pending = issue(0)
    for q in range(NCH):
        for c in pending:
            c.wait()
        if q + 1 < NCH:
            pending = issue(q + 1)
        b = q % 2

        def group(g, carry, _q=q, _b=b):
            rows = g * L + lanes
            gsl = pl.ds(_q * CH + g * L, L)
            hoff = ((hiv[gsl] >> 13) & 3) * D
            toff = ((tiv[gsl] >> 13) & 3) * D
            acc = jnp.zeros((L,), jnp.float32)
            for j in range(D):
                cols = jnp.full((L,), j, jnp.int32)
                hv = plsc.load_gather(hbuf.at[_b], [rows, hoff + j])
                rv = plsc.load_gather(rbuf.at[_b], [rows, cols])
                tv = plsc.load_gather(tbuf.at[_b], [rows, toff + j])
                d = (hv + rv) - tv
                acc = acc + d * d
            i32 = plsc.bitcast(acc, jnp.int32)
            y = plsc.bitcast(
                jnp.full((L,), 0x5F3759DF, jnp.int32) - (i32 >> 1),
                jnp.float32)
            for _ in range(3):
                y = y * (1.5 - 0.5 * ((acc * y) * y))
            outv[pl.ds(_q * CH + g * L, L)] = -(acc * y)
            return carry

        lax.fori_loop(0, GPC, group, 0)

    pltpu.sync_copy(outv, out_h.at[pl.ds(base, BPW)])


def kernel(entity_embeddings, relation_embeddings,
           head_indices, relation_indices, tail_indices):
    ent4 = _repack(entity_embeddings.T)
    relp = jnp.pad(relation_embeddings, ((0, 0), (0, PD - D)))
    return _transe_scores(ent4, relp,
                          head_indices, relation_indices, tail_indices)


# speedup vs baseline: 1.5910x; 1.0037x over previous
import functools

import jax
import jax.numpy as jnp
from jax import lax
from jax.experimental import pallas as pl
from jax.experimental.pallas import tpu as pltpu
from jax.experimental.pallas import tpu_sc as plsc

NUM_ENTITIES = 1000000
NUM_RELATIONS = 1000
D = 32
PD = 128
B = 16384
NC = 2
NS = 16
L = 16
NW = NC * NS
BPW = B // NW
CH = 128
NCH = BPW // CH
GPC = CH // L

EBLK = 32768
NBLK = -(-NUM_ENTITIES // EBLK)
RBLK = EBLK // 4


def _repack_body(x_ref, o_ref):
    outs = []
    for k in range(4):
        xk = x_ref[:, pl.ds((EBLK // 4) * k, EBLK // 4)]
        outs.append(xk.T)
    o_ref[...] = jnp.concatenate(outs, axis=1)


_repack = pl.pallas_call(
    _repack_body,
    grid=(NBLK,),
    in_specs=[pl.BlockSpec((D, EBLK), lambda i: (0, i))],
    out_specs=pl.BlockSpec((RBLK, PD), lambda i: (i, 0)),
    out_shape=jax.ShapeDtypeStruct((NBLK * RBLK, PD), jnp.float32),
)

_mesh = plsc.VectorSubcoreMesh(core_axis_name="c", subcore_axis_name="s")


@functools.partial(
    pl.kernel,
    out_type=jax.ShapeDtypeStruct((B,), jnp.float32),
    mesh=_mesh,
    compiler_params=pltpu.CompilerParams(
        needs_layout_passes=False,
    ),
    scratch_types=[
        pltpu.VMEM((BPW,), jnp.int32),
        pltpu.VMEM((BPW,), jnp.int32),
        pltpu.VMEM((BPW,), jnp.int32),
        pltpu.VMEM((NCH, CH), jnp.int32),
        pltpu.VMEM((NCH, CH), jnp.int32),
        pltpu.VMEM((NCH, CH), jnp.int32),
        pltpu.VMEM((2, CH, PD), jnp.float32),
        pltpu.VMEM((2, CH, PD), jnp.float32),
        pltpu.VMEM((2, CH, PD), jnp.float32),
        pltpu.VMEM((BPW,), jnp.float32),
        pltpu.SemaphoreType.DMA,
        pltpu.SemaphoreType.DMA,
    ],
)
def _transe_scores(ent4_h, relp_h, hi_h, ri_h, ti_h, out_h,
                   hiv, riv, tiv, hrow, rrow, trow,
                   hbuf, rbuf, tbuf, outv, sem0, sem1):
    wid = lax.axis_index("s") * NC + lax.axis_index("c")
    base = wid * BPW

    pltpu.sync_copy(hi_h.at[pl.ds(base, BPW)], hiv)
    pltpu.sync_copy(ri_h.at[pl.ds(base, BPW)], riv)
    pltpu.sync_copy(ti_h.at[pl.ds(base, BPW)], tiv)

    for q in range(NCH):
        for k in range(CH // L):
            sl = pl.ds(q * CH + k * L, L)
            dsl = pl.ds(k * L, L)
            he = hiv[sl]
            te = tiv[sl]
            hrow[q, dsl] = ((he >> 15) << 13) + (he & 8191)
            rrow[q, dsl] = riv[sl]
            trow[q, dsl] = ((te >> 15) << 13) + (te & 8191)

    sems = (sem0, sem1)

    def issue(q):
        s = sems[q % 2]
        b = q % 2
        return [
            pltpu.async_copy(ent4_h.at[hrow.at[q]], hbuf.at[b], s),
            pltpu.async_copy(ent4_h.at[trow.at[q]], tbuf.at[b], s),
            pltpu.async_copy(relp_h.at[rrow.at[q]], rbuf.at[b], s),
        ]

    lanes = lax.iota(jnp.int32, L)
    pending = issue(0)
    for q in range(NCH):
        for c in pending:
            c.wait()
        if q + 1 < NCH:
            pending = issue(q + 1)
        b = q % 2

        def group(g, carry, _q=q, _b=b):
            rows = g * L + lanes
            gsl = pl.ds(_q * CH + g * L, L)
            hoff = ((hiv[gsl] >> 13) & 3) * D
            toff = ((tiv[gsl] >> 13) & 3) * D
            acc = jnp.zeros((L,), jnp.float32)
            for j in range(D):
                cols = jnp.full((L,), j, jnp.int32)
                hv = plsc.load_gather(hbuf.at[_b], [rows, hoff + j])
                rv = plsc.load_gather(rbuf.at[_b], [rows, cols])
                tv = plsc.load_gather(tbuf.at[_b], [rows, toff + j])
                d = (hv + rv) - tv
                acc = acc + d * d
            i32 = plsc.bitcast(acc, jnp.int32)
            y = plsc.bitcast(
                jnp.full((L,), 0x5F3759DF, jnp.int32) - (i32 >> 1),
                jnp.float32)
            for _ in range(3):
                y = y * (1.5 - 0.5 * ((acc * y) * y))
            outv[pl.ds(_q * CH + g * L, L)] = -(acc * y)
            return carry

        lax.fori_loop(0, GPC, group, 0)

    pltpu.sync_copy(outv, out_h.at[pl.ds(base, BPW)])


def kernel(entity_embeddings, relation_embeddings,
           head_indices, relation_indices, tail_indices):
    ent4 = _repack(entity_embeddings.T)
    relp = jnp.pad(relation_embeddings, ((0, 0), (0, PD - D)))
    return _transe_scores(ent4, relp,
                          head_indices, relation_indices, tail_indices)
